# Initial kernel scaffold; baseline (speedup 1.0000x reference)
import jax
import jax.numpy as jnp
from jax.experimental import pallas as pl


def kernel(x, W_gcn, b_gcn):
    raise NotImplementedError("write your pallas kernel here")


# speedup vs baseline: 7.0167x; 7.0167x over previous
import functools

import jax
import jax.numpy as jnp
from jax import lax
from jax.experimental import pallas as pl

K_RATIO = 0.04
SIM_THRESHOLD = 0.6


def _cumsum_incl(v):
    hw = v.shape[1]
    d = 1
    while d < hw:
        padded = lax.pad(v, jnp.float32(0.0), ((0, 0, 0), (d, 0, 0)))
        shifted = lax.slice(padded, (0, 0), (1, hw))
        v = v + shifted
        d *= 2
    return v


def _onehot_dot(s, x, dims):
    sb = s.astype(jnp.bfloat16)
    hi = x.astype(jnp.bfloat16)
    r = x - hi.astype(jnp.float32)
    mid = r.astype(jnp.bfloat16)
    lo = (r - mid.astype(jnp.float32)).astype(jnp.bfloat16)
    acc = lax.dot_general(sb, hi, dims, preferred_element_type=jnp.float32)
    acc = acc + lax.dot_general(sb, mid, dims,
                                preferred_element_type=jnp.float32)
    acc = acc + lax.dot_general(sb, lo, dims,
                                preferred_element_type=jnp.float32)
    return acc


def _select_gather_body(nsel, nsp, x_ref, feats_ref, possel_ref):
    xb = x_ref[0]
    hw = xb.shape[1]
    score = jnp.mean(jnp.abs(xb), axis=0, keepdims=True)
    sbits = lax.bitcast_convert_type(score, jnp.int32)

    nself = jnp.float32(nsel)

    def bs_body(_, carry):
        lo, hi = carry
        mid = lo + ((hi - lo + 1) >> 1)
        cnt = jnp.sum((sbits >= mid).astype(jnp.float32))
        take = cnt >= nself
        return (jnp.where(take, mid, lo), jnp.where(take, hi, mid))

    lo, hi = lax.fori_loop(0, 31, bs_body,
                           (jnp.int32(0), jnp.int32(0x7F800000)))
    cnt_gt = jnp.sum((sbits > lo).astype(jnp.float32))

    eq = sbits == lo
    eqf = eq.astype(jnp.float32)
    rank_eq_excl = _cumsum_incl(eqf) - eqf
    sel = (sbits > lo) | (eq & (rank_eq_excl < (nself - cnt_gt)))
    self32 = sel.astype(jnp.float32)
    pos0 = _cumsum_incl(self32) - 1.0
    possel = jnp.where(sel, pos0.astype(jnp.int32), jnp.int32(-1))
    possel_ref[0] = possel

    niota = lax.broadcasted_iota(jnp.int32, (nsp, hw), 0)
    s_mat = (possel == niota).astype(jnp.float32)
    feats = _onehot_dot(s_mat, xb, (((1,), (1,)), ((), ())))
    feats_ref[0] = feats


def _graph_body(feats_ref, w_ref, b_ref, upd_ref):
    f = feats_ref[...]
    n2 = jnp.sum(f * f, axis=1, keepdims=True)
    nrm = f / (jnp.sqrt(n2) + 1e-12)
    sim = lax.dot_general(nrm, nrm, (((1,), (1,)), ((), ())),
                          preferred_element_type=jnp.float32,
                          precision=lax.Precision.HIGHEST)
    inv = (1.0 - sim) * 0.5
    thr = (1.0 - SIM_THRESHOLD) * 0.5
    adj = jnp.where(inv < thr, inv, 0.0)
    h = lax.dot_general(f, w_ref[...], (((1,), (0,)), ((), ())),
                        preferred_element_type=jnp.float32) + b_ref[...]
    upd = lax.dot_general(adj, h, (((1,), (0,)), ((), ())),
                          preferred_element_type=jnp.float32)
    upd_ref[...] = jnp.maximum(upd, 0.0)


def _scatter_body(nsp, x_ref, possel_ref, upd_ref, out_ref):
    xb = x_ref[0]
    ps = possel_ref[0][0:1]
    u = upd_ref[0]
    hw = xb.shape[1]
    niota = lax.broadcasted_iota(jnp.int32, (nsp, hw), 0)
    sb = (ps == niota).astype(jnp.bfloat16)
    dims = (((0,), (0,)), ((), ()))
    hi = u.astype(jnp.bfloat16)
    r = u - hi.astype(jnp.float32)
    mid = r.astype(jnp.bfloat16)
    lo = (r - mid.astype(jnp.float32)).astype(jnp.bfloat16)
    scat = lax.dot_general(hi, sb, dims, preferred_element_type=jnp.float32)
    scat = scat + lax.dot_general(mid, sb, dims,
                                  preferred_element_type=jnp.float32)
    scat = scat + lax.dot_general(lo, sb, dims,
                                  preferred_element_type=jnp.float32)
    out_ref[0] = jnp.where(ps >= 0, scat, xb)


def kernel(x, W_gcn, b_gcn):
    B, C, H, W = x.shape
    HW = H * W
    nsel = int(HW * K_RATIO)
    nsp = ((nsel + 7) // 8) * 8
    xf = x.reshape(B, C, HW)

    feats, possel = pl.pallas_call(
        functools.partial(_select_gather_body, nsel, nsp),
        grid=(B,),
        in_specs=[pl.BlockSpec((1, C, HW), lambda b: (b, 0, 0))],
        out_specs=[
            pl.BlockSpec((1, nsp, C), lambda b: (b, 0, 0)),
            pl.BlockSpec((1, 1, HW), lambda b: (b, 0, 0)),
        ],
        out_shape=[
            jax.ShapeDtypeStruct((B, nsp, C), jnp.float32),
            jax.ShapeDtypeStruct((B, 1, HW), jnp.int32),
        ],
    )(xf)

    upd = pl.pallas_call(
        _graph_body,
        out_shape=jax.ShapeDtypeStruct((B * nsp, C), jnp.float32),
    )(feats.reshape(B * nsp, C), W_gcn, b_gcn.reshape(1, C))

    out = pl.pallas_call(
        functools.partial(_scatter_body, nsp),
        grid=(B,),
        in_specs=[
            pl.BlockSpec((1, C, HW), lambda b: (b, 0, 0)),
            pl.BlockSpec((1, 1, HW), lambda b: (b, 0, 0)),
            pl.BlockSpec((1, nsp, C), lambda b: (b, 0, 0)),
        ],
        out_specs=pl.BlockSpec((1, C, HW), lambda b: (b, 0, 0)),
        out_shape=jax.ShapeDtypeStruct((B, C, HW), jnp.float32),
    )(xf, possel, upd.reshape(B, nsp, C))

    return out.reshape(B, C, H, W)
